# baseline (speedup 1.0000x reference)
import functools

import jax
import jax.numpy as jnp
from jax import lax
from jax.experimental import pallas as pl
from jax.experimental.pallas import tpu as pltpu
from jax.experimental.pallas import tpu_sc as plsc

NC = 2
NS = 16
L = 16
NW = NC * NS

BATCH = 16384
H_FEAT = 64
N_CLASSES = 2
BPW = BATCH // NW
NNODE = 1_000_000
SLABW = 512
NDC = 1953
TAIL0 = NDC * SLABW
NSLAB = 62
CAP = 1280
WIN = 128
DUMMY = 2 * BATCH
VROWS = 2 * BATCH + 8
SENTINEL = 1 << 29

_mesh = plsc.VectorSubcoreMesh(core_axis_name="c", subcore_axis_name="s")
_cparams = pltpu.CompilerParams(
    needs_layout_passes=False, use_tc_tiling_on_sc=True)


def _phase_a_body(embT_hbm, tail_hbm, ids_hbm, slots_hbm, starts_hbm,
                  out_hbm, ids_v, slots_v, tail_v, slab_v, colbuf, slotw,
                  starts_v, sem):
    wid = lax.axis_index("s") * NC + lax.axis_index("c")
    pltpu.sync_copy(starts_hbm, starts_v)
    pltpu.sync_copy(tail_hbm, tail_v)

    lane = lax.iota(jnp.int32, L)
    dummyvec = jnp.full((L,), DUMMY, jnp.int32)
    stc = (wid * NDC) // NW

    svec = jnp.where(lane == (wid % L),
                     starts_v[pl.ds((wid // L) * L, L)], 0)
    start = jnp.sum(svec)
    sa = pl.multiple_of((start // 8) * 8, 8)
    pltpu.sync_copy(ids_hbm.at[pl.ds(sa, CAP + 24)], ids_v)
    pltpu.sync_copy(slots_hbm.at[pl.ds(sa, CAP + 24)], slots_v)

    def initw(i, carry):
        slotw[0, pl.ds(i * L, L)] = dummyvec
        slotw[1, pl.ds(i * L, L)] = dummyvec
        return carry

    lax.fori_loop(0, WIN // L, initw, 0)

    def fire(c, slot):
        dc = jnp.minimum(stc + c, NDC - 1)
        pltpu.async_copy(embT_hbm.at[:, pl.ds(dc * SLABW, SLABW)],
                         slab_v.at[slot, :, pl.ds(0, SLABW)], sem)

    def drain(slot):
        pltpu.make_async_copy(embT_hbm.at[:, pl.ds(0, SLABW)],
                              slab_v.at[slot, :, pl.ds(0, SLABW)], sem).wait()

    def read_id(p):
        pa = (p // L) * L
        vec = ids_v[pl.ds(pa, L)]
        return jnp.sum(jnp.where(lane == (p % L), vec, 0))

    def note_slot(pp):
        pa = (pp // L) * L
        svals = slots_v[pl.ds(pa, L)]
        pr = pp % (2 * WIN)
        plsc.store_scatter(
            slotw,
            [jnp.full((L,), pr // WIN, jnp.int32),
             jnp.full((L,), pr % WIN, jnp.int32)],
            svals,
            mask=lane == (pp % L))

    def flush(pfx):
        win = pfx % (2 * WIN)
        pltpu.sync_copy(
            colbuf.at[pl.ds(pl.multiple_of(win, WIN), WIN)],
            out_hbm.at[slotw.at[win // WIN]])

        def rst(i, carry):
            plsc.store_scatter(
                slotw,
                [jnp.full((L,), win // WIN, jnp.int32), i * L + lane],
                dummyvec)
            return carry

        lax.fori_loop(0, WIN // L, rst, 0)

    def skcond(st):
        pp, idv = st
        return idv < stc * SLABW

    def skbody(st):
        pp, _ = st
        return pp + 1, read_id(pp + 1)

    p0, idv0 = lax.while_loop(skcond, skbody, (start - sa, read_id(start - sa)))
    pf0 = (p0 // WIN) * WIN

    def slab_join(c, carry):
        p, idv, pf = carry
        slot = c % 2
        fire(c + 1, (c + 1) % 2)
        drain(slot)
        slab_end = jnp.minimum(stc + c + 1, NDC) * SLABW

        def cond(st):
            pp, iv = st
            return iv < slab_end

        def body(st):
            pp, iv = st
            l = iv - (stc + c) * SLABW
            row = pp % (2 * WIN)
            for k in range(H_FEAT // L):
                ch = plsc.load_gather(
                    slab_v, [jnp.full((L,), slot, jnp.int32),
                             k * L + lane,
                             jnp.full((L,), l, jnp.int32)])
                colbuf[row, pl.ds(k * L, L)] = ch
            note_slot(pp)
            return pp + 1, read_id(pp + 1)

        p, idv = lax.while_loop(cond, body, (p, idv))

        def fcond(st):
            pp, pfx = st
            return pfx + WIN <= pp

        def fbody(st):
            pp, pfx = st
            flush(pfx)
            return pp, pfx + WIN

        p, pf = lax.while_loop(fcond, fbody, (p, pf))
        return p, idv, pf

    fire(0, 0)
    p, idv, pf = lax.fori_loop(0, NSLAB, slab_join, (p0, idv0, pf0))
    drain(NSLAB % 2)

    def tcond(st):
        pp, iv = st
        return jnp.logical_and(iv >= TAIL0, iv < SENTINEL)

    def tbody(st):
        pp, iv = st
        r = iv - TAIL0
        row = pp % (2 * WIN)
        for k in range(H_FEAT // L):
            ch = plsc.load_gather(
                tail_v, [jnp.full((L,), r, jnp.int32), k * L + lane])
            colbuf[row, pl.ds(k * L, L)] = ch
        note_slot(pp)
        return pp + 1, read_id(pp + 1)

    p, idv = lax.while_loop(tcond, tbody, (p, idv))

    @pl.when(pf < p)
    def _():
        flush(pf)

    @pl.when(pf + WIN < p)
    def _():
        flush(pf + WIN)


_phase_a = functools.partial(
    pl.kernel,
    out_type=jax.ShapeDtypeStruct((VROWS, 128), jnp.float32),
    mesh=_mesh,
    compiler_params=_cparams,
    scratch_types=[
        pltpu.VMEM((CAP + 24,), jnp.int32),
        pltpu.VMEM((CAP + 24,), jnp.int32),
        pltpu.VMEM((64, H_FEAT), jnp.float32),
        pltpu.VMEM((2, H_FEAT, SLABW + 1), jnp.float32),
        pltpu.VMEM((2 * WIN, 128), jnp.float32),
        pltpu.VMEM((2, WIN), jnp.int32),
        pltpu.VMEM((NW,), jnp.int32),
        pltpu.SemaphoreType.DMA,
    ],
)(_phase_a_body)


def _phase_b_body(vals_hbm, w_hbm, b_hbm, out_hbm,
                  chunk_v, wv, bv, outv, sem):
    wid = lax.axis_index("s") * NC + lax.axis_index("c")
    pltpu.sync_copy(w_hbm, wv)
    pltpu.sync_copy(b_hbm, bv)

    bvec = bv[...]
    nck = H_FEAT // L
    w0c = [wv[0, pl.ds(k * L, L)] for k in range(nck)]
    w1c = [wv[1, pl.ds(k * L, L)] for k in range(nck)]
    b0vec = jnp.full((L,), bvec[0], jnp.float32)
    b1vec = jnp.full((L,), bvec[1], jnp.float32)
    lane = lax.iota(jnp.int32, L)
    mask_hi = lane == (L - 1)

    NED = 128
    NCH = BPW // NED

    def fire(c, slot):
        cc = jnp.minimum(c, NCH - 1)
        pltpu.async_copy(
            vals_hbm.at[pl.ds(wid * 2 * BPW + cc * 2 * NED, 2 * NED)],
            chunk_v.at[slot], sem)

    def drain(slot):
        pltpu.make_async_copy(vals_hbm.at[pl.ds(0, 2 * NED)],
                              chunk_v.at[slot], sem).wait()

    def chunk(c, carry):
        slot = c % 2
        fire(c + 1, (c + 1) % 2)
        drain(slot)

        def grp(g, carry2):
            e0 = g * L
            for ln in range(L):
                r = 2 * (e0 + ln)
                t0 = None
                t1 = None
                for k in range(nck):
                    cd = (chunk_v[slot, r, pl.ds(k * L, L)]
                          * chunk_v[slot, r + 1, pl.ds(k * L, L)])
                    p0 = cd * w0c[k]
                    p1 = cd * w1c[k]
                    t0 = p0 if t0 is None else t0 + p0
                    t1 = p1 if t1 is None else t1 + p1
                cs0 = plsc.cumsum(t0) + b0vec
                cs1 = plsc.cumsum(t1) + b1vec
                oidx = jnp.full(
                    (L,), N_CLASSES * (c * NED + e0 + ln), jnp.int32)
                plsc.store_scatter(outv, [oidx], cs0, mask=mask_hi)
                plsc.store_scatter(outv, [oidx + 1], cs1, mask=mask_hi)
            return carry2

        lax.fori_loop(0, NED // L, grp, 0)
        return carry

    fire(0, 0)
    lax.fori_loop(0, NCH, chunk, 0)
    drain(NCH % 2)
    pltpu.sync_copy(outv, out_hbm.at[pl.ds(wid * BPW * N_CLASSES,
                                           BPW * N_CLASSES)])


_phase_b = functools.partial(
    pl.kernel,
    out_type=jax.ShapeDtypeStruct((BATCH * N_CLASSES,), jnp.float32),
    mesh=_mesh,
    compiler_params=_cparams,
    scratch_types=[
        pltpu.VMEM((2, 256, 128), jnp.float32),
        pltpu.VMEM((N_CLASSES, H_FEAT), jnp.float32),
        pltpu.VMEM((L,), jnp.float32),
        pltpu.VMEM((BATCH * N_CLASSES // NW,), jnp.float32),
        pltpu.SemaphoreType.DMA,
    ],
)(_phase_b_body)


def kernel(src_id, dst_id, embedding, W, b):
    src_id = src_id.astype(jnp.int32)
    dst_id = dst_id.astype(jnp.int32)
    ar = jnp.arange(BATCH, dtype=jnp.int32)
    ids_all = jnp.concatenate([src_id, dst_id])
    slots_all = jnp.concatenate([2 * ar, 2 * ar + 1])
    ids_s, slots_s = lax.sort([ids_all, slots_all], num_keys=1)

    ids_ext = jnp.concatenate(
        [ids_s, jnp.full((CAP + 24,), SENTINEL, jnp.int32)])
    slots_ext = jnp.concatenate(
        [slots_s, jnp.full((CAP + 24,), DUMMY, jnp.int32)])

    bounds = ((jnp.arange(NW, dtype=jnp.int32) * NDC) // NW) * SLABW
    starts = jnp.searchsorted(
        ids_s, bounds, method="compare_all").astype(jnp.int32)

    tail_rows = lax.slice(embedding, (TAIL0, 0), (NNODE, H_FEAT))
    vals = _phase_a(embedding.T, tail_rows, ids_ext, slots_ext, starts)
    b_pad = jnp.zeros((L,), jnp.float32).at[:N_CLASSES].set(b)
    out_flat = _phase_b(vals, W, b_pad)
    return out_flat.reshape(BATCH, N_CLASSES)


# speedup vs baseline: 1.0000x; 1.0000x over previous
import functools

import jax
import jax.numpy as jnp
from jax import lax
from jax.experimental import pallas as pl
from jax.experimental.pallas import tpu as pltpu
from jax.experimental.pallas import tpu_sc as plsc

NC = 2
NS = 16
L = 16
NW = NC * NS

BATCH = 16384
H_FEAT = 64
N_CLASSES = 2
BPW = BATCH // NW
NNODE = 1_000_000
SLABW = 512
NDC = 1953
TAIL0 = NDC * SLABW
NSLAB = 62
CAP = 1280
WIN = 128
DUMMY = 2 * BATCH
VROWS = 2 * BATCH + 8
SENTINEL = 1 << 29

_mesh = plsc.VectorSubcoreMesh(core_axis_name="c", subcore_axis_name="s")
_cparams = pltpu.CompilerParams(
    needs_layout_passes=False, use_tc_tiling_on_sc=True)


def _phase_a_body(embT_hbm, tail_hbm, ids_hbm, slots_hbm, starts_hbm,
                  out_hbm, ids_v, slots_v, tail_v, slab_v, colbuf, slotw,
                  starts_v, sem):
    wid = lax.axis_index("s") * NC + lax.axis_index("c")
    pltpu.sync_copy(starts_hbm, starts_v)
    pltpu.sync_copy(tail_hbm, tail_v)

    lane = lax.iota(jnp.int32, L)
    dummyvec = jnp.full((L,), DUMMY, jnp.int32)
    stc = (wid * NDC) // NW

    svec = jnp.where(lane == (wid % L),
                     starts_v[pl.ds((wid // L) * L, L)], 0)
    start = jnp.sum(svec)
    sa = pl.multiple_of((start // 8) * 8, 8)
    pltpu.sync_copy(ids_hbm.at[pl.ds(sa, CAP + 24)], ids_v)
    pltpu.sync_copy(slots_hbm.at[pl.ds(sa, CAP + 24)], slots_v)

    def initw(i, carry):
        slotw[0, pl.ds(i * L, L)] = dummyvec
        slotw[1, pl.ds(i * L, L)] = dummyvec
        return carry

    lax.fori_loop(0, WIN // L, initw, 0)

    def fire(c, slot):
        dc = jnp.minimum(stc + c, NDC - 1)
        for tr in range(8):
            pltpu.async_copy(
                embT_hbm.at[pl.ds(tr * 8, 8), pl.ds(dc * SLABW, SLABW)],
                slab_v.at[slot, pl.ds(tr * 8, 8), pl.ds(0, SLABW)], sem)

    def drain(slot):
        for tr in range(8):
            pltpu.make_async_copy(
                embT_hbm.at[pl.ds(0, 8), pl.ds(0, SLABW)],
                slab_v.at[slot, pl.ds(tr * 8, 8), pl.ds(0, SLABW)],
                sem).wait()

    def read_id(p):
        pa = (p // L) * L
        vec = ids_v[pl.ds(pa, L)]
        return jnp.sum(jnp.where(lane == (p % L), vec, 0))

    def note_slot(pp):
        pa = (pp // L) * L
        svals = slots_v[pl.ds(pa, L)]
        pr = pp % (2 * WIN)
        plsc.store_scatter(
            slotw,
            [jnp.full((L,), pr // WIN, jnp.int32),
             jnp.full((L,), pr % WIN, jnp.int32)],
            svals,
            mask=lane == (pp % L))

    def flush(pfx):
        win = pfx % (2 * WIN)
        pltpu.sync_copy(
            colbuf.at[pl.ds(pl.multiple_of(win, WIN), WIN)],
            out_hbm.at[slotw.at[win // WIN]])

        def rst(i, carry):
            plsc.store_scatter(
                slotw,
                [jnp.full((L,), win // WIN, jnp.int32), i * L + lane],
                dummyvec)
            return carry

        lax.fori_loop(0, WIN // L, rst, 0)

    def skcond(st):
        pp, idv = st
        return idv < stc * SLABW

    def skbody(st):
        pp, _ = st
        return pp + 1, read_id(pp + 1)

    p0, idv0 = lax.while_loop(skcond, skbody, (start - sa, read_id(start - sa)))
    pf0 = (p0 // WIN) * WIN

    def slab_join(c, carry):
        p, idv, pf = carry
        slot = c % 2
        fire(c + 1, (c + 1) % 2)
        drain(slot)
        slab_end = jnp.minimum(stc + c + 1, NDC) * SLABW

        def cond(st):
            pp, iv = st
            return iv < slab_end

        def body(st):
            pp, iv = st
            l = iv - (stc + c) * SLABW
            row = pp % (2 * WIN)
            for k in range(H_FEAT // L):
                ch = plsc.load_gather(
                    slab_v, [jnp.full((L,), slot, jnp.int32),
                             k * L + lane,
                             jnp.full((L,), l, jnp.int32)])
                colbuf[row, pl.ds(k * L, L)] = ch
            note_slot(pp)
            return pp + 1, read_id(pp + 1)

        p, idv = lax.while_loop(cond, body, (p, idv))

        def fcond(st):
            pp, pfx = st
            return pfx + WIN <= pp

        def fbody(st):
            pp, pfx = st
            flush(pfx)
            return pp, pfx + WIN

        p, pf = lax.while_loop(fcond, fbody, (p, pf))
        return p, idv, pf

    fire(0, 0)
    p, idv, pf = lax.fori_loop(0, NSLAB, slab_join, (p0, idv0, pf0))
    drain(NSLAB % 2)

    def tcond(st):
        pp, iv = st
        return jnp.logical_and(iv >= TAIL0, iv < SENTINEL)

    def tbody(st):
        pp, iv = st
        r = iv - TAIL0
        row = pp % (2 * WIN)
        for k in range(H_FEAT // L):
            ch = plsc.load_gather(
                tail_v, [jnp.full((L,), r, jnp.int32), k * L + lane])
            colbuf[row, pl.ds(k * L, L)] = ch
        note_slot(pp)
        return pp + 1, read_id(pp + 1)

    p, idv = lax.while_loop(tcond, tbody, (p, idv))

    @pl.when(pf < p)
    def _():
        flush(pf)

    @pl.when(pf + WIN < p)
    def _():
        flush(pf + WIN)


_phase_a = functools.partial(
    pl.kernel,
    out_type=jax.ShapeDtypeStruct((VROWS, 128), jnp.float32),
    mesh=_mesh,
    compiler_params=_cparams,
    scratch_types=[
        pltpu.VMEM((CAP + 24,), jnp.int32),
        pltpu.VMEM((CAP + 24,), jnp.int32),
        pltpu.VMEM((64, H_FEAT), jnp.float32),
        pltpu.VMEM((2, H_FEAT, SLABW + 1), jnp.float32),
        pltpu.VMEM((2 * WIN, 128), jnp.float32),
        pltpu.VMEM((2, WIN), jnp.int32),
        pltpu.VMEM((NW,), jnp.int32),
        pltpu.SemaphoreType.DMA,
    ],
)(_phase_a_body)


def _phase_b_body(vals_hbm, w_hbm, b_hbm, out_hbm,
                  chunk_v, wv, bv, outv, sem):
    wid = lax.axis_index("s") * NC + lax.axis_index("c")
    pltpu.sync_copy(w_hbm, wv)
    pltpu.sync_copy(b_hbm, bv)

    bvec = bv[...]
    nck = H_FEAT // L
    w0c = [wv[0, pl.ds(k * L, L)] for k in range(nck)]
    w1c = [wv[1, pl.ds(k * L, L)] for k in range(nck)]
    b0vec = jnp.full((L,), bvec[0], jnp.float32)
    b1vec = jnp.full((L,), bvec[1], jnp.float32)
    lane = lax.iota(jnp.int32, L)
    mask_hi = lane == (L - 1)

    NED = 128
    NCH = BPW // NED

    def fire(c, slot):
        cc = jnp.minimum(c, NCH - 1)
        pltpu.async_copy(
            vals_hbm.at[pl.ds(wid * 2 * BPW + cc * 2 * NED, 2 * NED)],
            chunk_v.at[slot], sem)

    def drain(slot):
        pltpu.make_async_copy(vals_hbm.at[pl.ds(0, 2 * NED)],
                              chunk_v.at[slot], sem).wait()

    def chunk(c, carry):
        slot = c % 2
        fire(c + 1, (c + 1) % 2)
        drain(slot)

        def grp(g, carry2):
            e0 = g * L
            for ln in range(L):
                r = 2 * (e0 + ln)
                t0 = None
                t1 = None
                for k in range(nck):
                    cd = (chunk_v[slot, r, pl.ds(k * L, L)]
                          * chunk_v[slot, r + 1, pl.ds(k * L, L)])
                    p0 = cd * w0c[k]
                    p1 = cd * w1c[k]
                    t0 = p0 if t0 is None else t0 + p0
                    t1 = p1 if t1 is None else t1 + p1
                cs0 = plsc.cumsum(t0) + b0vec
                cs1 = plsc.cumsum(t1) + b1vec
                oidx = jnp.full(
                    (L,), N_CLASSES * (c * NED + e0 + ln), jnp.int32)
                plsc.store_scatter(outv, [oidx], cs0, mask=mask_hi)
                plsc.store_scatter(outv, [oidx + 1], cs1, mask=mask_hi)
            return carry2

        lax.fori_loop(0, NED // L, grp, 0)
        return carry

    fire(0, 0)
    lax.fori_loop(0, NCH, chunk, 0)
    drain(NCH % 2)
    pltpu.sync_copy(outv, out_hbm.at[pl.ds(wid * BPW * N_CLASSES,
                                           BPW * N_CLASSES)])


_phase_b = functools.partial(
    pl.kernel,
    out_type=jax.ShapeDtypeStruct((BATCH * N_CLASSES,), jnp.float32),
    mesh=_mesh,
    compiler_params=_cparams,
    scratch_types=[
        pltpu.VMEM((2, 256, 128), jnp.float32),
        pltpu.VMEM((N_CLASSES, H_FEAT), jnp.float32),
        pltpu.VMEM((L,), jnp.float32),
        pltpu.VMEM((BATCH * N_CLASSES // NW,), jnp.float32),
        pltpu.SemaphoreType.DMA,
    ],
)(_phase_b_body)


def kernel(src_id, dst_id, embedding, W, b):
    src_id = src_id.astype(jnp.int32)
    dst_id = dst_id.astype(jnp.int32)
    ar = jnp.arange(BATCH, dtype=jnp.int32)
    ids_all = jnp.concatenate([src_id, dst_id])
    slots_all = jnp.concatenate([2 * ar, 2 * ar + 1])
    ids_s, slots_s = lax.sort([ids_all, slots_all], num_keys=1)

    ids_ext = jnp.concatenate(
        [ids_s, jnp.full((CAP + 24,), SENTINEL, jnp.int32)])
    slots_ext = jnp.concatenate(
        [slots_s, jnp.full((CAP + 24,), DUMMY, jnp.int32)])

    bounds = ((jnp.arange(NW, dtype=jnp.int32) * NDC) // NW) * SLABW
    starts = jnp.searchsorted(
        ids_s, bounds, method="compare_all").astype(jnp.int32)

    tail_rows = lax.slice(embedding, (TAIL0, 0), (NNODE, H_FEAT))
    vals = _phase_a(embedding.T, tail_rows, ids_ext, slots_ext, starts)
    b_pad = jnp.zeros((L,), jnp.float32).at[:N_CLASSES].set(b)
    out_flat = _phase_b(vals, W, b_pad)
    return out_flat.reshape(BATCH, N_CLASSES)


# speedup vs baseline: 1.0635x; 1.0635x over previous
import functools

import jax
import jax.numpy as jnp
from jax import lax
from jax.experimental import pallas as pl
from jax.experimental.pallas import tpu as pltpu
from jax.experimental.pallas import tpu_sc as plsc

NC = 2
NS = 16
L = 16
NW = NC * NS

BATCH = 16384
H_FEAT = 64
N_CLASSES = 2
BPW = BATCH // NW
NNODE = 1_000_000
SLABW = 512
NDC = 1953
TAIL0 = NDC * SLABW
NSLAB = 62
CAP = 1280
WIN = 128
DUMMY = 2 * BATCH
VROWS = 2 * BATCH + 8
SENTINEL = 1 << 29

_mesh = plsc.VectorSubcoreMesh(core_axis_name="c", subcore_axis_name="s")
_cparams = pltpu.CompilerParams(
    needs_layout_passes=False, use_tc_tiling_on_sc=True)


def _phase_a_body(embT_hbm, tail_hbm, ids_hbm, slots_hbm, starts_hbm,
                  out_hbm, ids_v, slots_v, tail_v, slab_v, colbuf, slotw,
                  starts_v, sem):
    wid = lax.axis_index("s") * NC + lax.axis_index("c")
    pltpu.sync_copy(starts_hbm, starts_v)
    pltpu.sync_copy(tail_hbm, tail_v)

    lane = lax.iota(jnp.int32, L)
    dummyvec = jnp.full((L,), DUMMY, jnp.int32)
    stc = (wid * NDC) // NW

    svec = jnp.where(lane == (wid % L),
                     starts_v[pl.ds((wid // L) * L, L)], 0)
    start = jnp.sum(svec)
    sa = pl.multiple_of((start // 8) * 8, 8)
    pltpu.sync_copy(ids_hbm.at[pl.ds(sa, CAP + 24)], ids_v)
    pltpu.sync_copy(slots_hbm.at[pl.ds(sa, CAP + 24)], slots_v)

    def initw(i, carry):
        slotw[0, pl.ds(i * L, L)] = dummyvec
        slotw[1, pl.ds(i * L, L)] = dummyvec
        return carry

    lax.fori_loop(0, WIN // L, initw, 0)

    def fire(c, slot):
        dc = jnp.minimum(stc + c, NDC - 1)
        for tr in range(8):
            pltpu.async_copy(
                embT_hbm.at[pl.ds(tr * 8, 8), pl.ds(dc * SLABW, SLABW)],
                slab_v.at[slot, pl.ds(tr * 8, 8), pl.ds(0, SLABW)], sem)

    def drain(slot):
        for tr in range(8):
            pltpu.make_async_copy(
                embT_hbm.at[pl.ds(0, 8), pl.ds(0, SLABW)],
                slab_v.at[slot, pl.ds(tr * 8, 8), pl.ds(0, SLABW)],
                sem).wait()

    def read_id(p):
        pa = (p // L) * L
        vec = ids_v[pl.ds(pa, L)]
        return jnp.sum(jnp.where(lane == (p % L), vec, 0))

    def note_slot(pp):
        pa = (pp // L) * L
        svals = slots_v[pl.ds(pa, L)]
        pr = pp % (2 * WIN)
        plsc.store_scatter(
            slotw,
            [jnp.full((L,), pr // WIN, jnp.int32),
             jnp.full((L,), pr % WIN, jnp.int32)],
            svals,
            mask=lane == (pp % L))

    def flush(pfx):
        win = pfx % (2 * WIN)
        pltpu.sync_copy(
            colbuf.at[pl.ds(pl.multiple_of(win, WIN), WIN)],
            out_hbm.at[slotw.at[win // WIN]])

        def rst(i, carry):
            plsc.store_scatter(
                slotw,
                [jnp.full((L,), win // WIN, jnp.int32), i * L + lane],
                dummyvec)
            return carry

        lax.fori_loop(0, WIN // L, rst, 0)

    def skcond(st):
        pp, idv = st
        return idv < stc * SLABW

    def skbody(st):
        pp, _ = st
        return pp + 1, read_id(pp + 1)

    p0, idv0 = lax.while_loop(skcond, skbody, (start - sa, read_id(start - sa)))
    pf0 = (p0 // WIN) * WIN

    def slab_join(c, carry):
        p, idv, pf = carry
        slot = c % 2
        fire(c + 1, (c + 1) % 2)
        drain(slot)
        slab_end = jnp.minimum(stc + c + 1, NDC) * SLABW

        def cond(st):
            pp, iv = st
            return iv < slab_end

        def body(st):
            pp, iv = st
            l = iv - (stc + c) * SLABW
            row = pp % (2 * WIN)
            for k in range(H_FEAT // L):
                ch = plsc.load_gather(
                    slab_v, [jnp.full((L,), slot, jnp.int32),
                             k * L + lane,
                             jnp.full((L,), l, jnp.int32)])
                colbuf[row, pl.ds(k * L, L)] = ch
            note_slot(pp)
            return pp + 1, read_id(pp + 1)

        p, idv = lax.while_loop(cond, body, (p, idv))

        def fcond(st):
            pp, pfx = st
            return pfx + WIN <= pp

        def fbody(st):
            pp, pfx = st
            flush(pfx)
            return pp, pfx + WIN

        p, pf = lax.while_loop(fcond, fbody, (p, pf))
        return p, idv, pf

    fire(0, 0)
    p, idv, pf = lax.fori_loop(0, NSLAB, slab_join, (p0, idv0, pf0))
    drain(NSLAB % 2)

    def tcond(st):
        pp, iv = st
        return jnp.logical_and(iv >= TAIL0, iv < SENTINEL)

    def tbody(st):
        pp, iv = st
        r = iv - TAIL0
        row = pp % (2 * WIN)
        for k in range(H_FEAT // L):
            ch = plsc.load_gather(
                tail_v, [jnp.full((L,), r, jnp.int32), k * L + lane])
            colbuf[row, pl.ds(k * L, L)] = ch
        note_slot(pp)
        return pp + 1, read_id(pp + 1)

    p, idv = lax.while_loop(tcond, tbody, (p, idv))

    @pl.when(pf < p)
    def _():
        flush(pf)

    @pl.when(pf + WIN < p)
    def _():
        flush(pf + WIN)


_phase_a = functools.partial(
    pl.kernel,
    out_type=jax.ShapeDtypeStruct((VROWS, 128), jnp.float32),
    mesh=_mesh,
    compiler_params=_cparams,
    scratch_types=[
        pltpu.VMEM((CAP + 24,), jnp.int32),
        pltpu.VMEM((CAP + 24,), jnp.int32),
        pltpu.VMEM((64, H_FEAT), jnp.float32),
        pltpu.VMEM((2, H_FEAT, SLABW + 1), jnp.float32),
        pltpu.VMEM((2 * WIN, 128), jnp.float32),
        pltpu.VMEM((2, WIN), jnp.int32),
        pltpu.VMEM((NW,), jnp.int32),
        pltpu.SemaphoreType.DMA,
    ],
)(_phase_a_body)


def _phase_b_body(vals_hbm, w_hbm, b_hbm, out_hbm,
                  chunk_v, wv, bv, outv, sem):
    wid = lax.axis_index("s") * NC + lax.axis_index("c")
    pltpu.sync_copy(w_hbm, wv)
    pltpu.sync_copy(b_hbm, bv)

    bvec = bv[...]
    nck = H_FEAT // L
    w0c = [wv[0, pl.ds(k * L, L)] for k in range(nck)]
    w1c = [wv[1, pl.ds(k * L, L)] for k in range(nck)]
    b0vec = jnp.full((L,), bvec[0], jnp.float32)
    b1vec = jnp.full((L,), bvec[1], jnp.float32)
    lane = lax.iota(jnp.int32, L)
    mask_hi = lane == (L - 1)

    NED = 128
    NCH = BPW // NED

    def fire(c, slot):
        cc = jnp.minimum(c, NCH - 1)
        pltpu.async_copy(
            vals_hbm.at[pl.ds(wid * 2 * BPW + cc * 2 * NED, 2 * NED)],
            chunk_v.at[slot], sem)

    def drain(slot):
        pltpu.make_async_copy(vals_hbm.at[pl.ds(0, 2 * NED)],
                              chunk_v.at[slot], sem).wait()

    def chunk(c, carry):
        slot = c % 2
        fire(c + 1, (c + 1) % 2)
        drain(slot)

        def grp(g, carry2):
            e0 = g * L
            for ln in range(L):
                r = 2 * (e0 + ln)
                t0 = None
                t1 = None
                for k in range(nck):
                    cd = (chunk_v[slot, r, pl.ds(k * L, L)]
                          * chunk_v[slot, r + 1, pl.ds(k * L, L)])
                    p0 = cd * w0c[k]
                    p1 = cd * w1c[k]
                    t0 = p0 if t0 is None else t0 + p0
                    t1 = p1 if t1 is None else t1 + p1
                cs0 = plsc.cumsum(t0) + b0vec
                cs1 = plsc.cumsum(t1) + b1vec
                oidx = jnp.full((L,), c * NED + e0 + ln, jnp.int32)
                plsc.store_scatter(outv, [oidx], cs0, mask=mask_hi)
                plsc.store_scatter(outv, [oidx + BPW], cs1, mask=mask_hi)
            return carry2

        lax.fori_loop(0, NED // L, grp, 0)
        return carry

    fire(0, 0)
    lax.fori_loop(0, NCH, chunk, 0)
    drain(NCH % 2)
    pltpu.sync_copy(outv.at[pl.ds(0, BPW)],
                    out_hbm.at[0, pl.ds(wid * BPW, BPW)])
    pltpu.sync_copy(outv.at[pl.ds(BPW, BPW)],
                    out_hbm.at[1, pl.ds(wid * BPW, BPW)])


_phase_b = functools.partial(
    pl.kernel,
    out_type=jax.ShapeDtypeStruct((N_CLASSES, BATCH), jnp.float32),
    mesh=_mesh,
    compiler_params=_cparams,
    scratch_types=[
        pltpu.VMEM((2, 256, 128), jnp.float32),
        pltpu.VMEM((N_CLASSES, H_FEAT), jnp.float32),
        pltpu.VMEM((L,), jnp.float32),
        pltpu.VMEM((BATCH * N_CLASSES // NW,), jnp.float32),
        pltpu.SemaphoreType.DMA,
    ],
)(_phase_b_body)


def kernel(src_id, dst_id, embedding, W, b):
    src_id = src_id.astype(jnp.int32)
    dst_id = dst_id.astype(jnp.int32)
    ar = jnp.arange(BATCH, dtype=jnp.int32)
    ids_all = jnp.concatenate([src_id, dst_id])
    slots_all = jnp.concatenate([2 * ar, 2 * ar + 1])
    ids_s, slots_s = lax.sort([ids_all, slots_all], num_keys=1)

    ids_ext = jnp.concatenate(
        [ids_s, jnp.full((CAP + 24,), SENTINEL, jnp.int32)])
    slots_ext = jnp.concatenate(
        [slots_s, jnp.full((CAP + 24,), DUMMY, jnp.int32)])

    bounds = ((jnp.arange(NW, dtype=jnp.int32) * NDC) // NW) * SLABW
    starts = jnp.searchsorted(
        ids_s, bounds, method="compare_all").astype(jnp.int32)

    tail_rows = lax.slice(embedding, (TAIL0, 0), (NNODE, H_FEAT))
    vals = _phase_a(embedding.T, tail_rows, ids_ext, slots_ext, starts)
    b_pad = jnp.zeros((L,), jnp.float32).at[:N_CLASSES].set(b)
    out_cm = _phase_b(vals, W, b_pad)
    return out_cm.T
